# speedup vs baseline: 9.2678x; 9.2678x over previous
import functools

import jax
import jax.numpy as jnp
from jax.experimental import pallas as pl

_N = 4096
_MAX_PED = 16
_MIN_R = 0.5
_MAX_R = 10.0
_BR = 256


def _body(xrow, yrow, xcol, ycol, hxcol, hycol, idxcol, radcol, out_ref):
    xi = xcol[...]
    yi = ycol[...]
    hx = hxcol[...]
    hy = hycol[...]
    sqh = hx * hx + hy * hy
    xj = xrow[...]
    yj = yrow[...]
    dx = xj - xi
    dy = yj - yi
    squ = dx * dx + dy * dy
    dot = dx * hx + dy * hy
    in_sight = (dot > 0.0) & (4.0 * (dot * dot) > squ * sqh)
    inf = jnp.float32(jnp.inf)
    msk = jnp.where(in_sight, squ, inf)
    n_in_sight = jnp.sum(in_sight.astype(jnp.int32), axis=1, keepdims=True)

    def step(_, carry):
        msk, s15, s16, cum = carry
        m = jnp.min(msk, axis=1, keepdims=True)
        hit = msk == m
        c = jnp.sum(hit.astype(jnp.int32), axis=1, keepdims=True)
        nxt = cum + c
        s15 = jnp.where((cum <= _MAX_PED - 1) & (nxt > _MAX_PED - 1), m, s15)
        s16 = jnp.where((cum <= _MAX_PED) & (nxt > _MAX_PED), m, s16)
        msk = jnp.where(hit, inf, msk)
        return msk, s15, s16, nxt

    zero_i = jnp.zeros((_BR, 1), jnp.int32)
    init = (msk, jnp.full((_BR, 1), inf), jnp.full((_BR, 1), inf), zero_i)
    _, s15, s16, _ = jax.lax.fori_loop(0, _MAX_PED + 1, step, init)

    r = 0.5 * (jnp.sqrt(s15) + jnp.sqrt(s16))
    r = jnp.clip(r, _MIN_R, _MAX_R)
    r = jnp.where(n_in_sight <= _MAX_PED, jnp.float32(_MAX_R), r)
    out_ref[...] = jnp.where(idxcol[...] != 0, r, radcol[...])


def kernel(past_ped_positions, ped_positions, indexes, all_radii):
    n = ped_positions.shape[0]
    x = ped_positions[:, 0]
    y = ped_positions[:, 1]
    hx = x - past_ped_positions[:, 0]
    hy = y - past_ped_positions[:, 1]
    xrow = x.reshape(1, n)
    yrow = y.reshape(1, n)
    xcol = x.reshape(n, 1)
    ycol = y.reshape(n, 1)
    hxcol = hx.reshape(n, 1)
    hycol = hy.reshape(n, 1)
    idxcol = indexes.astype(jnp.int32).reshape(n, 1)
    radcol = all_radii.reshape(n, 1)

    grid = (n // _BR,)
    row_spec = pl.BlockSpec((1, n), lambda i: (0, 0))
    col_spec = pl.BlockSpec((_BR, 1), lambda i: (i, 0))
    out = pl.pallas_call(
        _body,
        grid=grid,
        in_specs=[row_spec, row_spec, col_spec, col_spec, col_spec, col_spec,
                  col_spec, col_spec],
        out_specs=col_spec,
        out_shape=jax.ShapeDtypeStruct((n, 1), jnp.float32),
    )(xrow, yrow, xcol, ycol, hxcol, hycol, idxcol, radcol)
    return out.reshape(n)


# speedup vs baseline: 15.9372x; 1.7196x over previous
import functools

import jax
import jax.numpy as jnp
from jax.experimental import pallas as pl

_N = 4096
_MAX_PED = 16
_MIN_R = 0.5
_MAX_R = 10.0
_BR = 256


def _body(xrow, yrow, xcol, ycol, hxcol, hycol, idxcol, radcol, out_ref):
    xi = xcol[...]
    yi = ycol[...]
    hx = hxcol[...]
    hy = hycol[...]
    sqh = hx * hx + hy * hy
    xj = xrow[...]
    yj = yrow[...]
    dx = xj - xi
    dy = yj - yi
    squ = dx * dx + dy * dy
    dot = dx * hx + dy * hy
    in_sight = (dot > 0.0) & (4.0 * (dot * dot) > squ * sqh)
    inf = jnp.float32(jnp.inf)
    msk = jnp.where(in_sight, squ, inf)
    n_in_sight = jnp.sum(in_sight.astype(jnp.int32), axis=1, keepdims=True)

    def step(_, carry):
        m, ms = carry
        nxt = jnp.min(jnp.where(msk > m, msk, inf), axis=1, keepdims=True)
        return nxt, ms + [nxt]

    m = jnp.full((_BR, 1), -jnp.inf, jnp.float32)
    ms = []
    for _ in range(_MAX_PED + 1):
        m, ms = step(None, (m, ms))

    s15 = jnp.full((_BR, 1), inf)
    s16 = jnp.full((_BR, 1), inf)
    cum = jnp.zeros((_BR, 1), jnp.int32)
    for k in range(_MAX_PED + 1):
        c = jnp.sum((msk == ms[k]).astype(jnp.int32), axis=1, keepdims=True)
        nxt = cum + c
        s15 = jnp.where((cum <= _MAX_PED - 1) & (nxt > _MAX_PED - 1), ms[k], s15)
        s16 = jnp.where((cum <= _MAX_PED) & (nxt > _MAX_PED), ms[k], s16)
        cum = nxt

    r = 0.5 * (jnp.sqrt(s15) + jnp.sqrt(s16))
    r = jnp.clip(r, _MIN_R, _MAX_R)
    r = jnp.where(n_in_sight <= _MAX_PED, jnp.float32(_MAX_R), r)
    out_ref[...] = jnp.where(idxcol[...] != 0, r, radcol[...])


def kernel(past_ped_positions, ped_positions, indexes, all_radii):
    n = ped_positions.shape[0]
    x = ped_positions[:, 0]
    y = ped_positions[:, 1]
    hx = x - past_ped_positions[:, 0]
    hy = y - past_ped_positions[:, 1]
    xrow = x.reshape(1, n)
    yrow = y.reshape(1, n)
    xcol = x.reshape(n, 1)
    ycol = y.reshape(n, 1)
    hxcol = hx.reshape(n, 1)
    hycol = hy.reshape(n, 1)
    idxcol = indexes.astype(jnp.int32).reshape(n, 1)
    radcol = all_radii.reshape(n, 1)

    grid = (n // _BR,)
    row_spec = pl.BlockSpec((1, n), lambda i: (0, 0))
    col_spec = pl.BlockSpec((_BR, 1), lambda i: (i, 0))
    out = pl.pallas_call(
        _body,
        grid=grid,
        in_specs=[row_spec, row_spec, col_spec, col_spec, col_spec, col_spec,
                  col_spec, col_spec],
        out_specs=col_spec,
        out_shape=jax.ShapeDtypeStruct((n, 1), jnp.float32),
    )(xrow, yrow, xcol, ycol, hxcol, hycol, idxcol, radcol)
    return out.reshape(n)
